# baseline (speedup 1.0000x reference)
import jax
import jax.numpy as jnp
from jax import lax
from jax.experimental import pallas as pl
from jax.experimental.pallas import tpu as pltpu
from jax.experimental.pallas import tpu_sc as plsc

_NC = 2
_NS = 16
_NW = _NC * _NS
_L = 16
_D = 32
_FB = 4
_B = 16384
_BPW = _B // _NW
_NCHUNK = 4
_CH = _BPW // _NCHUNK
_V = 1000000
_REG = 0.001


def _sc_body(rows_hbm, cols_hbm, p3, q3, preds_hbm, regs_hbm,
             idx_r, idx_c, pe, qe, out_v, reg_v, sem_p, sem_q):
    wid = lax.axis_index("s") * _NC + lax.axis_index("c")
    pltpu.sync_copy(rows_hbm.at[wid], idx_r)
    pltpu.sync_copy(cols_hbm.at[wid], idx_c)

    def start_f(f):
        fb = f // 8
        s = f % 8
        for c in range(_NCHUNK):
            pltpu.async_copy(
                p3.at[fb].at[s].at[idx_r.at[c]], pe.at[f].at[c], sem_p)
            pltpu.async_copy(
                q3.at[fb].at[s].at[idx_c.at[c]], qe.at[f].at[c], sem_q)

    def wait_f(f):
        fb = f // 8
        s = f % 8
        for c in range(_NCHUNK):
            pltpu.make_async_copy(
                p3.at[fb].at[s].at[idx_r.at[c]], pe.at[f].at[c], sem_p).wait()
            pltpu.make_async_copy(
                q3.at[fb].at[s].at[idx_c.at[c]], qe.at[f].at[c], sem_q).wait()

    start_f(0)

    def gather_step(f, _):
        start_f_dyn(f)
        wait_f_dyn(f - 1)
        return 0

    def start_f_dyn(f):
        fb = lax.div(f, 8)
        s = lax.rem(f, 8)
        for c in range(_NCHUNK):
            pltpu.async_copy(
                p3.at[fb].at[s].at[idx_r.at[c]], pe.at[f].at[c], sem_p)
            pltpu.async_copy(
                q3.at[fb].at[s].at[idx_c.at[c]], qe.at[f].at[c], sem_q)

    def wait_f_dyn(f):
        fb = lax.div(f, 8)
        s = lax.rem(f, 8)
        for c in range(_NCHUNK):
            pltpu.make_async_copy(
                p3.at[fb].at[s].at[idx_r.at[c]], pe.at[f].at[c], sem_p).wait()
            pltpu.make_async_copy(
                q3.at[fb].at[s].at[idx_c.at[c]], qe.at[f].at[c], sem_q).wait()

    lax.fori_loop(1, _D, gather_step, 0)
    wait_f(_D - 1)

    def group(g, carry):
        accp, accq = carry
        c = g // 8
        b = (g % 8) * _L
        acc = jnp.zeros((_L,), jnp.float32)
        for f in range(_D):
            pv = pe[f, c, pl.ds(b, _L)]
            qv = qe[f, c, pl.ds(b, _L)]
            acc = acc + pv * qv
            accp = accp + pv * pv
            accq = accq + qv * qv
        out_v[pl.ds(g * _L, _L)] = acc
        return accp, accq

    zero = jnp.zeros((_L,), jnp.float32)
    accp, accq = lax.fori_loop(0, _BPW // _L, group, (zero, zero))
    reg_v[0] = accp * _REG
    reg_v[1] = accq * _REG

    pltpu.sync_copy(out_v, preds_hbm.at[wid])
    pltpu.sync_copy(reg_v, regs_hbm.at[wid])


@jax.jit
def kernel(rows, cols, ratval, P, Q):
    del ratval
    rows3 = rows.reshape(_NW, _NCHUNK, _CH)
    cols3 = cols.reshape(_NW, _NCHUNK, _CH)
    p3 = jnp.swapaxes(P, 0, 1).reshape(_FB, 8, _V)
    q3 = jnp.swapaxes(Q, 0, 1).reshape(_FB, 8, _V)
    mesh = plsc.VectorSubcoreMesh(core_axis_name="c", subcore_axis_name="s")
    run = pl.kernel(
        _sc_body,
        out_type=[
            jax.ShapeDtypeStruct((_NW, _BPW), jnp.float32),
            jax.ShapeDtypeStruct((_NW, 2, _L), jnp.float32),
        ],
        mesh=mesh,
        compiler_params=pltpu.CompilerParams(
            needs_layout_passes=False,
            use_tc_tiling_on_sc=False,
        ),
        scratch_types=[
            pltpu.VMEM((_NCHUNK, _CH), jnp.int32),
            pltpu.VMEM((_NCHUNK, _CH), jnp.int32),
            pltpu.VMEM((_D, _NCHUNK, _CH), jnp.float32),
            pltpu.VMEM((_D, _NCHUNK, _CH), jnp.float32),
            pltpu.VMEM((_BPW,), jnp.float32),
            pltpu.VMEM((2, _L), jnp.float32),
            pltpu.SemaphoreType.DMA,
            pltpu.SemaphoreType.DMA,
        ],
    )
    preds, regs = run(rows3, cols3, p3, q3)
    preds_rat = preds.reshape(_B, 1)
    ues_reg = jnp.sum(regs[:, 0, :])
    uis_rat_reg = jnp.sum(regs[:, 1, :])
    return (preds_rat, ues_reg, uis_rat_reg)


# speedup vs baseline: 1.0001x; 1.0001x over previous
import jax
import jax.numpy as jnp
from jax import lax
from jax.experimental import pallas as pl
from jax.experimental.pallas import tpu as pltpu
from jax.experimental.pallas import tpu_sc as plsc

_NC = 2
_NS = 16
_NW = _NC * _NS
_L = 16
_D = 32
_FB = 4
_B = 16384
_BPW = _B // _NW
_NCHUNK = 4
_CH = _BPW // _NCHUNK
_V = 1000000
_REG = 0.001


def _sc_body(rows_hbm, cols_hbm, p3, q3, preds_hbm, regs_hbm,
             idx_r, idx_c, pe, qe, out_v, reg_v, sem_p, sem_q):
    wid = lax.axis_index("s") * _NC + lax.axis_index("c")
    pltpu.sync_copy(rows_hbm.at[wid], idx_r)
    pltpu.sync_copy(cols_hbm.at[wid], idx_c)

    def start_f(f):
        for c in range(_NCHUNK):
            pltpu.async_copy(
                p3.at[f].at[idx_r.at[c]], pe.at[f].at[c], sem_p)
            pltpu.async_copy(
                q3.at[f].at[idx_c.at[c]], qe.at[f].at[c], sem_q)

    def wait_f(f):
        for c in range(_NCHUNK):
            pltpu.make_async_copy(
                p3.at[f].at[idx_r.at[c]], pe.at[f].at[c], sem_p).wait()
            pltpu.make_async_copy(
                q3.at[f].at[idx_c.at[c]], qe.at[f].at[c], sem_q).wait()

    start_f(0)

    def gather_step(f, _):
        start_f(f)
        wait_f(f - 1)
        return 0

    lax.fori_loop(1, _D, gather_step, 0)
    wait_f(_D - 1)

    def group(g, carry):
        accp, accq = carry
        c = g // 8
        b = (g % 8) * _L
        acc = jnp.zeros((_L,), jnp.float32)
        for f in range(_D):
            pv = pe[f, c, pl.ds(b, _L)]
            qv = qe[f, c, pl.ds(b, _L)]
            acc = acc + pv * qv
            accp = accp + pv * pv
            accq = accq + qv * qv
        out_v[pl.ds(g * _L, _L)] = acc
        return accp, accq

    zero = jnp.zeros((_L,), jnp.float32)
    accp, accq = lax.fori_loop(0, _BPW // _L, group, (zero, zero))
    reg_v[0] = accp * _REG
    reg_v[1] = accq * _REG

    pltpu.sync_copy(out_v, preds_hbm.at[wid])
    pltpu.sync_copy(reg_v, regs_hbm.at[wid])


@jax.jit
def kernel(rows, cols, ratval, P, Q):
    del ratval
    rows3 = rows.reshape(_NW, _NCHUNK, _CH)
    cols3 = cols.reshape(_NW, _NCHUNK, _CH)
    p3 = jnp.swapaxes(P, 0, 1)
    q3 = jnp.swapaxes(Q, 0, 1)
    mesh = plsc.VectorSubcoreMesh(core_axis_name="c", subcore_axis_name="s")
    run = pl.kernel(
        _sc_body,
        out_type=[
            jax.ShapeDtypeStruct((_NW, _BPW), jnp.float32),
            jax.ShapeDtypeStruct((_NW, 2, _L), jnp.float32),
        ],
        mesh=mesh,
        compiler_params=pltpu.CompilerParams(
            needs_layout_passes=False,
            use_tc_tiling_on_sc=False,
        ),
        scratch_types=[
            pltpu.VMEM((_NCHUNK, _CH), jnp.int32),
            pltpu.VMEM((_NCHUNK, _CH), jnp.int32),
            pltpu.VMEM((_D, _NCHUNK, _CH), jnp.float32),
            pltpu.VMEM((_D, _NCHUNK, _CH), jnp.float32),
            pltpu.VMEM((_BPW,), jnp.float32),
            pltpu.VMEM((2, _L), jnp.float32),
            pltpu.SemaphoreType.DMA,
            pltpu.SemaphoreType.DMA,
        ],
    )
    preds, regs = run(rows3, cols3, p3, q3)
    preds_rat = preds.reshape(_B, 1)
    ues_reg = jnp.sum(regs[:, 0, :])
    uis_rat_reg = jnp.sum(regs[:, 1, :])
    return (preds_rat, ues_reg, uis_rat_reg)
